# baseline (speedup 1.0000x reference)
import functools

import jax
import jax.numpy as jnp
from jax import lax
from jax.experimental import pallas as pl
from jax.experimental.pallas import tpu as pltpu
from jax.experimental.pallas import tpu_sc as plsc

_B = 8
_R = 1024
_C = 512
_D = 64
_NROWS = _B * 4


def _sc_gather(table, idx1, idx2, num_cores):
    mesh = plsc.VectorSubcoreMesh(core_axis_name="c", subcore_axis_name="s")

    @functools.partial(
        pl.kernel,
        mesh=mesh,
        compiler_params=pltpu.CompilerParams(needs_layout_passes=False),
        out_type=jax.ShapeDtypeStruct((_NROWS * 128,), jnp.float32),
        scratch_types=[
            pltpu.VMEM((2, 2), jnp.int32),
            pltpu.VMEM((32,), jnp.int32),
            pltpu.VMEM((_D, 128), jnp.float32),
            pltpu.VMEM((_D,), jnp.float32),
            pltpu.SemaphoreType.DMA,
            pltpu.SemaphoreType.DMA,
        ],
    )
    def k(table_hbm, i1_hbm, i2_hbm, out_hbm, i1_v, i2_v, rows_v, out_v, s1, s2):
        t = lax.axis_index("s") * num_cores + lax.axis_index("c")
        cp1 = pltpu.async_copy(i1_hbm, i1_v, s1)
        cp2 = pltpu.async_copy(i2_hbm, i2_v.at[pl.ds(0, 2)], s2)
        cp1.wait()
        cp2.wait()
        iv = jnp.broadcast_to((t >> 1) & 1, (16,))
        jv = jnp.broadcast_to(t & 1, (16,))
        r_s = plsc.load_gather(i1_v, [iv, jv])[0]
        c_s = i2_v[pl.ds(t & 1, 16)][0]
        base = ((t >> 2) * _R + r_s) * _D
        ctile = (c_s >> 7) * 128
        pltpu.sync_copy(
            table_hbm.at[pl.ds(base, _D), pl.ds(ctile, 128)], rows_v
        )
        lane = lax.broadcasted_iota(jnp.int32, (16,), 0)
        coff = jnp.broadcast_to(c_s & 127, (16,))
        for kk in range(4):
            out_v[pl.ds(kk * 16, 16)] = plsc.load_gather(
                rows_v, [kk * 16 + lane, coff]
            )
        pltpu.sync_copy(out_v, out_hbm.at[pl.ds(t * 128, _D)])

    return k(table, idx1, idx2)


def kernel(x, index1, index2):
    table = x.transpose(0, 1, 3, 2).reshape(_B * _R * _D, _C)
    num_cores = plsc.get_sparse_core_info().num_cores
    out = _sc_gather(table, index1, index2, num_cores)
    return out.reshape(_NROWS, 128)[:, :_D].reshape(_B, 2, 2, _D)


# speedup vs baseline: 1.0858x; 1.0858x over previous
import functools

import jax
import jax.numpy as jnp
from jax import lax
from jax.experimental import pallas as pl
from jax.experimental.pallas import tpu as pltpu
from jax.experimental.pallas import tpu_sc as plsc

_B = 8
_R = 1024
_C = 512
_D = 64
_NROWS = _B * 4


def _sc_gather(table, idx1, idx2):
    mesh = plsc.VectorSubcoreMesh(
        core_axis_name="c", subcore_axis_name="s", num_cores=1
    )

    @functools.partial(
        pl.kernel,
        mesh=mesh,
        compiler_params=pltpu.CompilerParams(needs_layout_passes=False),
        out_type=jax.ShapeDtypeStruct((_NROWS * 128,), jnp.float32),
        scratch_types=[
            pltpu.VMEM((2, 2), jnp.int32),
            pltpu.VMEM((32,), jnp.int32),
            pltpu.VMEM((2, _D, 128), jnp.float32),
            pltpu.VMEM((256,), jnp.float32),
            pltpu.SemaphoreType.DMA,
            pltpu.SemaphoreType.DMA,
        ],
    )
    def k(table_hbm, i1_hbm, i2_hbm, out_hbm, i1_v, i2_v, rows_v, out_v, s1, s2):
        t = lax.axis_index("s")
        cp1 = pltpu.async_copy(i1_hbm, i1_v, s1)
        cp2 = pltpu.async_copy(i2_hbm, i2_v.at[pl.ds(0, 2)], s2)
        cp1.wait()
        cp2.wait()
        copies = []
        coffs = []
        for h in range(2):
            combo = t * 2 + h
            iv = jnp.broadcast_to((combo >> 1) & 1, (16,))
            jv = jnp.broadcast_to(combo & 1, (16,))
            r_s = plsc.load_gather(i1_v, [iv, jv])[0]
            c_s = i2_v[pl.ds(h, 16)][0]
            base = ((combo >> 2) * _R + r_s) * _D
            ctile = (c_s >> 7) * 128
            copies.append(
                pltpu.async_copy(
                    table_hbm.at[pl.ds(base, _D), pl.ds(ctile, 128)],
                    rows_v.at[h],
                    s1 if h == 0 else s2,
                )
            )
            coffs.append(c_s & 127)
        lane = lax.broadcasted_iota(jnp.int32, (16,), 0)
        for h in range(2):
            copies[h].wait()
            coff = jnp.broadcast_to(coffs[h], (16,))
            for kk in range(4):
                out_v[pl.ds(h * 128 + kk * 16, 16)] = plsc.load_gather(
                    rows_v, [jnp.broadcast_to(h, (16,)), kk * 16 + lane, coff]
                )
        pltpu.sync_copy(out_v, out_hbm.at[pl.ds(t * 256, 256)])

    return k(table, idx1, idx2)


def kernel(x, index1, index2):
    table = x.transpose(0, 1, 3, 2).reshape(_B * _R * _D, _C)
    out = _sc_gather(table, index1, index2)
    return out.reshape(_NROWS, 128)[:, :_D].reshape(_B, 2, 2, _D)
